# baseline (speedup 1.0000x reference)
import functools

import jax
import jax.numpy as jnp
from jax import lax
from jax.experimental import pallas as pl
from jax.experimental.pallas import tpu as pltpu
from jax.experimental.pallas import tpu_sc as plsc

VOCAB = 100000
N_HIDDEN = 128
B = 16384
F = 26
TOT = B * F
NW = 32
ROWS_W = B // NW
HALF = ROWS_W // 2
LANES = 16
XW = 32

VB = 12800
TC_GRID = (VOCAB + VB - 1) // VB
XB = B // TC_GRID


VB_TAIL = VOCAB - (TC_GRID - 1) * VB


def _project_body(e_ref, w_ref, b_ref, x_ref, o_ref, xp_ref):
    i = pl.program_id(0)
    e = e_ref[...]
    w = w_ref[...]
    row = (
        lax.dot_general(w, e, (((1,), (1,)), ((), ())),
                        preferred_element_type=jnp.float32)
        + b_ref[0, 0]
    ).reshape(VB)

    @pl.when(i < TC_GRID - 1)
    def _():
        o_ref[pl.ds(i * VB, VB)] = row

    @pl.when(i == TC_GRID - 1)
    def _():
        o_ref[pl.ds((TC_GRID - 1) * VB, VB_TAIL)] = row[:VB_TAIL]

    x = x_ref[...]
    xp_ref[...] = jnp.concatenate(
        [x, jnp.zeros((XB, 128 - F), jnp.int32)], axis=1
    )


def _project_table(emb, W, b2d, x):
    return pl.pallas_call(
        _project_body,
        grid=(TC_GRID,),
        in_specs=[
            pl.BlockSpec((VB, N_HIDDEN), lambda i: (i, 0)),
            pl.BlockSpec((1, N_HIDDEN), lambda i: (0, 0)),
            pl.BlockSpec((1, 1), lambda i: (0, 0)),
            pl.BlockSpec((XB, F), lambda i: (i, 0)),
        ],
        out_specs=[
            pl.BlockSpec((VOCAB,), lambda i: (0,)),
            pl.BlockSpec((XB, 128), lambda i: (i, 0)),
        ],
        out_shape=[
            jax.ShapeDtypeStruct((VOCAB,), jnp.float32),
            jax.ShapeDtypeStruct((B, 128), jnp.int32),
        ],
    )(emb, W, b2d, x)


def _unpad_body(y_ref, o_ref):
    o_ref[...] = y_ref[:, :F]


def _unpad(yp):
    return pl.pallas_call(
        _unpad_body,
        grid=(TC_GRID,),
        in_specs=[pl.BlockSpec((XB, 128), lambda i: (i, 0))],
        out_specs=pl.BlockSpec((XB, F), lambda i: (i, 0)),
        out_shape=jax.ShapeDtypeStruct((B, F), jnp.float32),
    )(yp)


def _sc_gather_body(v_hbm, xp_hbm, yp_hbm, v_v, xin_v, out_v, sem_v, sem_i):
    wid = lax.axis_index("s") * 2 + lax.axis_index("c")
    base = wid * ROWS_W
    cp_v = pltpu.async_copy(v_hbm, v_v, sem_v)
    cp0 = pltpu.async_copy(
        xp_hbm.at[pl.ds(base, HALF), pl.ds(0, XW)], xin_v, sem_i
    )
    cp_v.wait()

    def run_half(h):
        @plsc.parallel_loop(0, HALF, step=1, unroll=4)
        def per_row(r):
            for k in range(2):
                idx = xin_v[r, pl.ds(k * LANES, LANES)]
                out_v[h * HALF + r, pl.ds(k * LANES, LANES)] = (
                    plsc.load_gather(v_v, [idx])
                )

    cp0.wait()
    run_half(0)
    cp1 = pltpu.async_copy(
        xp_hbm.at[pl.ds(base + HALF, HALF), pl.ds(0, XW)], xin_v, sem_i
    )
    cp1.wait()
    run_half(1)
    pltpu.sync_copy(out_v, yp_hbm.at[pl.ds(base, ROWS_W), pl.ds(0, XW)])


@functools.cache
def _sc_gather():
    mesh = plsc.VectorSubcoreMesh(core_axis_name="c", subcore_axis_name="s")
    return pl.kernel(
        _sc_gather_body,
        out_type=jax.ShapeDtypeStruct((B, 128), jnp.float32),
        mesh=mesh,
        scratch_types=[
            pltpu.VMEM((VOCAB,), jnp.float32),
            pltpu.VMEM((HALF, XW), jnp.int32),
            pltpu.VMEM((ROWS_W, XW), jnp.float32),
            pltpu.SemaphoreType.DMA,
            pltpu.SemaphoreType.DMA,
        ],
        compiler_params=pltpu.CompilerParams(
            needs_layout_passes=False, use_tc_tiling_on_sc=False
        ),
    )


def kernel(x, emb, W, b):
    v, xp = _project_table(emb, W, b.reshape(1, 1), x)
    yp = _sc_gather()(v, xp)
    return _unpad(yp).reshape(B, F, 1)


# speedup vs baseline: 1.4510x; 1.4510x over previous
import functools

import jax
import jax.numpy as jnp
from jax import lax
from jax.experimental import pallas as pl
from jax.experimental.pallas import tpu as pltpu
from jax.experimental.pallas import tpu_sc as plsc

VOCAB = 100000
N_HIDDEN = 128
B = 16384
F = 26
TOT = B * F
NW = 32
COLS_W = B // NW
LANES = 16

VB = 12800
TC_GRID = (VOCAB + VB - 1) // VB
VB_TAIL = VOCAB - (TC_GRID - 1) * VB
XB = B // TC_GRID
FXB = F * XB


def _project_body(e_ref, w_ref, b_ref, xt_ref, o_ref, xf_ref):
    i = pl.program_id(0)
    e = e_ref[...]
    w = w_ref[...]
    row = (
        lax.dot_general(w, e, (((1,), (1,)), ((), ())),
                        preferred_element_type=jnp.float32)
        + b_ref[0, 0]
    ).reshape(VB)

    @pl.when(i < TC_GRID - 1)
    def _():
        o_ref[pl.ds(i * VB, VB)] = row

    @pl.when(i == TC_GRID - 1)
    def _():
        o_ref[pl.ds((TC_GRID - 1) * VB, VB_TAIL)] = row[:VB_TAIL]

    xf_ref[...] = xt_ref[...].reshape(FXB)


def _project_table(emb, W, b2d, xt):
    return pl.pallas_call(
        _project_body,
        grid=(TC_GRID,),
        in_specs=[
            pl.BlockSpec((VB, N_HIDDEN), lambda i: (i, 0)),
            pl.BlockSpec((1, N_HIDDEN), lambda i: (0, 0)),
            pl.BlockSpec((1, 1), lambda i: (0, 0)),
            pl.BlockSpec((F, XB), lambda i: (0, i)),
        ],
        out_specs=[
            pl.BlockSpec((VOCAB,), lambda i: (0,)),
            pl.BlockSpec((FXB,), lambda i: (i,)),
        ],
        out_shape=[
            jax.ShapeDtypeStruct((VOCAB,), jnp.float32),
            jax.ShapeDtypeStruct((TC_GRID * FXB,), jnp.int32),
        ],
    )(emb, W, b2d, xt)


def _sc_gather_body(v_hbm, xf_hbm, yt_hbm, v_v, xin_v, out_v, sem_v, sem_i):
    wid = lax.axis_index("s") * 2 + lax.axis_index("c")
    g = wid // (XB // COLS_W)
    c0 = (wid % (XB // COLS_W)) * COLS_W
    base = wid * COLS_W
    cp_v = pltpu.async_copy(v_hbm, v_v, sem_v)
    cp_i = pltpu.async_copy(
        xf_hbm.at[g, pl.ds(0, F), pl.ds(c0, COLS_W)], xin_v, sem_i
    )
    cp_v.wait()
    cp_i.wait()

    @plsc.parallel_loop(0, F * (COLS_W // LANES), step=1, unroll=4)
    def per_chunk(i):
        f = i // (COLS_W // LANES)
        c = (i - f * (COLS_W // LANES)) * LANES
        idx = xin_v[f, pl.ds(c, LANES)]
        out_v[f, pl.ds(c, LANES)] = plsc.load_gather(v_v, [idx])

    pltpu.sync_copy(out_v, yt_hbm.at[pl.ds(0, F), pl.ds(base, COLS_W)])


@functools.cache
def _sc_gather():
    mesh = plsc.VectorSubcoreMesh(core_axis_name="c", subcore_axis_name="s")
    return pl.kernel(
        _sc_gather_body,
        out_type=jax.ShapeDtypeStruct((F, B), jnp.float32),
        mesh=mesh,
        scratch_types=[
            pltpu.VMEM((VOCAB,), jnp.float32),
            pltpu.VMEM((F, COLS_W), jnp.int32),
            pltpu.VMEM((F, COLS_W), jnp.float32),
            pltpu.SemaphoreType.DMA,
            pltpu.SemaphoreType.DMA,
        ],
        compiler_params=pltpu.CompilerParams(
            needs_layout_passes=False, use_tc_tiling_on_sc=False
        ),
    )


def kernel(x, emb, W, b):
    v, xf = _project_table(emb, W, b.reshape(1, 1), x.T)
    yt = _sc_gather()(v, xf.reshape(TC_GRID, F, XB))
    return yt.T[:, :, None]


# speedup vs baseline: 1.4654x; 1.0100x over previous
import functools

import jax
import jax.numpy as jnp
from jax import lax
from jax.experimental import pallas as pl
from jax.experimental.pallas import tpu as pltpu
from jax.experimental.pallas import tpu_sc as plsc

VOCAB = 100000
N_HIDDEN = 128
B = 16384
F = 26
TOT = B * F
NW = 32
COLS_W = B // NW
LANES = 16

VB = 12800
TC_GRID = (VOCAB + VB - 1) // VB
VB_TAIL = VOCAB - (TC_GRID - 1) * VB
XB = B // TC_GRID
FXB = F * XB


def _project_body(e_ref, w_ref, b_ref, xt_ref, o_ref, xf_ref):
    i = pl.program_id(0)
    e = e_ref[...]
    w = w_ref[...]
    row = (
        lax.dot_general(w, e, (((1,), (1,)), ((), ())),
                        preferred_element_type=jnp.float32)
        + b_ref[0, 0]
    ).reshape(VB)

    @pl.when(i < TC_GRID - 1)
    def _():
        o_ref[pl.ds(i * VB, VB)] = row

    @pl.when(i == TC_GRID - 1)
    def _():
        o_ref[pl.ds((TC_GRID - 1) * VB, VB_TAIL)] = row[:VB_TAIL]

    xf_ref[...] = xt_ref[...].reshape(FXB)


def _project_table(emb, W, b2d, xt):
    return pl.pallas_call(
        _project_body,
        grid=(TC_GRID,),
        in_specs=[
            pl.BlockSpec((VB, N_HIDDEN), lambda i: (i, 0)),
            pl.BlockSpec((1, N_HIDDEN), lambda i: (0, 0)),
            pl.BlockSpec((1, 1), lambda i: (0, 0)),
            pl.BlockSpec((F, XB), lambda i: (0, i)),
        ],
        out_specs=[
            pl.BlockSpec((VOCAB,), lambda i: (0,)),
            pl.BlockSpec((FXB,), lambda i: (i,)),
        ],
        out_shape=[
            jax.ShapeDtypeStruct((VOCAB,), jnp.float32),
            jax.ShapeDtypeStruct((TC_GRID * FXB,), jnp.int32),
        ],
    )(emb, W, b2d, xt)


def _sc_gather_body(v_hbm, xf_hbm, yt_hbm, v_v, xin_v, out_v, sem_v, sem_i):
    wid = lax.axis_index("s") * 2 + lax.axis_index("c")
    g = wid // (XB // COLS_W)
    c0 = (wid % (XB // COLS_W)) * COLS_W
    base = wid * COLS_W
    cp_v = pltpu.async_copy(v_hbm, v_v, sem_v)
    cp_i = pltpu.async_copy(
        xf_hbm.at[g, pl.ds(0, F), pl.ds(c0, COLS_W)], xin_v, sem_i
    )
    cp_v.wait()
    cp_i.wait()

    CW = COLS_W // LANES
    FH = F // 2

    def gather_rows(f0, nf):
        @plsc.parallel_loop(0, nf * CW, step=1, unroll=8)
        def per_chunk(i):
            f = f0 + i // CW
            c = (i % CW) * LANES
            idx = xin_v[f, pl.ds(c, LANES)]
            out_v[f, pl.ds(c, LANES)] = plsc.load_gather(v_v, [idx])

    gather_rows(0, FH)
    cp_o = pltpu.async_copy(
        out_v.at[pl.ds(0, FH)], yt_hbm.at[pl.ds(0, FH), pl.ds(base, COLS_W)],
        sem_i,
    )
    gather_rows(FH, F - FH)
    pltpu.sync_copy(
        out_v.at[pl.ds(FH, F - FH)],
        yt_hbm.at[pl.ds(FH, F - FH), pl.ds(base, COLS_W)],
    )
    cp_o.wait()


@functools.cache
def _sc_gather():
    mesh = plsc.VectorSubcoreMesh(core_axis_name="c", subcore_axis_name="s")
    return pl.kernel(
        _sc_gather_body,
        out_type=jax.ShapeDtypeStruct((F, B), jnp.float32),
        mesh=mesh,
        scratch_types=[
            pltpu.VMEM((VOCAB,), jnp.float32),
            pltpu.VMEM((F, COLS_W), jnp.int32),
            pltpu.VMEM((F, COLS_W), jnp.float32),
            pltpu.SemaphoreType.DMA,
            pltpu.SemaphoreType.DMA,
        ],
        compiler_params=pltpu.CompilerParams(
            needs_layout_passes=False, use_tc_tiling_on_sc=False
        ),
    )


def kernel(x, emb, W, b):
    v, xf = _project_table(emb, W, b.reshape(1, 1), x.T)
    yt = _sc_gather()(v, xf.reshape(TC_GRID, F, XB))
    return lax.reshape(yt, (B, F, 1), dimensions=(1, 0))
